# baseline (speedup 1.0000x reference)
import functools

import jax
import jax.numpy as jnp
from jax import lax
from jax.experimental import pallas as pl
from jax.experimental.pallas import tpu as pltpu
from jax.experimental.pallas import tpu_sc as plsc


def _tc_transpose(table_t, n_rows, wide_rows):
    n_blocks = (n_rows + 1023) // 1024

    def body(in_ref, out_ref):
        x = in_ref[...]
        for k in range(4):
            out_ref[:, k * 32:(k + 1) * 32] = x[:, k * 256:(k + 1) * 256].T

    return pl.pallas_call(
        body,
        grid=(n_blocks,),
        in_specs=[pl.BlockSpec((32, 1024), lambda i: (0, i))],
        out_specs=pl.BlockSpec((256, 128), lambda i: (i, 0)),
        out_shape=jax.ShapeDtypeStruct((wide_rows, 128), jnp.float32),
    )(table_t)


def _make_sc_gather(n_rows_total, n_cond, chunk, n_chunks_per_worker,
                    n_workers, n_cores):
    mesh = plsc.VectorSubcoreMesh(core_axis_name="c", subcore_axis_name="s")
    rows_per_worker = chunk * n_chunks_per_worker
    epw = 2 * rows_per_worker
    epc = 2 * chunk
    phase_step = chunk % n_cond

    @functools.partial(
        pl.kernel,
        out_type=jax.ShapeDtypeStruct((2 * n_rows_total, 16), jnp.float32),
        mesh=mesh,
        scratch_types=[
            pltpu.VMEM((epw // 128, 128), jnp.int32),
            pltpu.VMEM(((n_cond + chunk) * 32,), jnp.float32),
            pltpu.VMEM((epc, 16), jnp.float32),
            pltpu.SemaphoreType.DMA,
        ],
        compiler_params=pltpu.CompilerParams(use_tc_tiling_on_sc=False),
    )
    def sc_kernel(ids_hbm, table_hbm, pat_hbm, out_hbm, idx_v, pat_v, gbuf,
                  sem):
        wid = lax.axis_index("s") * n_cores + lax.axis_index("c")
        pltpu.sync_copy(ids_hbm.at[wid], idx_v)
        pltpu.sync_copy(pat_hbm, pat_v)

        def chunk_body(g, carry):
            d1 = pltpu.async_copy(table_hbm.at[idx_v.at[2 * g]],
                                  gbuf.at[pl.ds(0, 128)], sem)
            d2 = pltpu.async_copy(table_hbm.at[idx_v.at[2 * g + 1]],
                                  gbuf.at[pl.ds(128, 128)], sem)
            d1.wait()
            d2.wait()

            p = lax.rem(g * phase_step, n_cond) * 32

            def add_body(rr, c2):
                plsc.addupdate(gbuf.at[rr, pl.ds(0, 16)],
                               pat_v[pl.ds(p + rr * 16, 16)])
                return c2

            lax.fori_loop(0, epc, add_body, 0, unroll=8)

            pltpu.sync_copy(gbuf, out_hbm.at[pl.ds(wid * epw + g * epc, epc)])
            return carry

        lax.fori_loop(0, n_chunks_per_worker, chunk_body, 0)

    return sc_kernel


def kernel(cat_ids, cond_table, cat_table):
    b, n_cond = cat_ids.shape
    dim = cat_table.shape[1]
    n_rows_table = cat_table.shape[0]
    n_cat = n_rows_table // n_cond

    info = plsc.get_sparse_core_info()
    n_cores, n_subcores = info.num_cores, info.num_subcores
    n_workers = n_cores * n_subcores

    n_rows = b * n_cond
    chunk = 128
    rows_per_worker = n_rows // n_workers
    n_chunks_per_worker = rows_per_worker // chunk
    assert rows_per_worker % chunk == 0

    wide_rows = 256 * ((n_rows_table + 1023) // 1024)

    wide = _tc_transpose(cat_table.T, n_rows_table, wide_rows)
    tbl16 = wide.reshape(wide_rows * 8, 16)

    offsets = jnp.arange(n_cond, dtype=jnp.int32) * n_cat
    r = (cat_ids.astype(jnp.int32) + offsets[None, :]).reshape(-1)
    e0 = ((r >> 10) * 256 + (r & 255)) * 8 + ((r >> 8) & 3) * 2
    ids_dbl = (e0[:, None] + jnp.arange(2, dtype=jnp.int32)[None, :]).reshape(
        n_workers, 2 * rows_per_worker // 128, 128)

    reps = (n_cond + chunk + n_cond - 1) // n_cond
    pat = jnp.tile(cond_table[1:n_cond + 1],
                   (reps, 1)).reshape(-1)[:(n_cond + chunk) * dim]

    sc_gather = _make_sc_gather(n_rows, n_cond, chunk, n_chunks_per_worker,
                                n_workers, n_cores)
    out = sc_gather(ids_dbl, tbl16, pat)
    return out.reshape(b, n_cond, dim)


# speedup vs baseline: 1.8893x; 1.8893x over previous
import functools

import jax
import jax.numpy as jnp
from jax import lax
from jax.experimental import pallas as pl
from jax.experimental.pallas import tpu as pltpu
from jax.experimental.pallas import tpu_sc as plsc


_TBLK = 8192


def _tc_transpose(table_t, n_rows, wide_rows):
    n_blocks = (n_rows + _TBLK - 1) // _TBLK
    wpb = _TBLK // 4

    def body(in_ref, out_ref):
        x = in_ref[...]
        eye = jnp.float32(
            lax.broadcasted_iota(jnp.int32, (32, 32), 0) ==
            lax.broadcasted_iota(jnp.int32, (32, 32), 1))
        y = lax.dot_general(x, eye, (((0,), (0,)), ((), ())),
                            preferred_element_type=jnp.float32)
        for k in range(4):
            out_ref[:, k * 32:(k + 1) * 32] = y[k * wpb:(k + 1) * wpb, :]

    return pl.pallas_call(
        body,
        grid=(n_blocks,),
        in_specs=[pl.BlockSpec((32, _TBLK), lambda i: (0, i))],
        out_specs=pl.BlockSpec((wpb, 128), lambda i: (i, 0)),
        out_shape=jax.ShapeDtypeStruct((wide_rows, 128), jnp.float32),
    )(table_t)


def _make_sc_gather(n_rows_total, n_cond, chunk, n_chunks_per_worker,
                    n_workers, n_cores):
    mesh = plsc.VectorSubcoreMesh(core_axis_name="c", subcore_axis_name="s")
    rows_per_worker = chunk * n_chunks_per_worker
    epw = 2 * rows_per_worker
    epc = 2 * chunk
    phase_step = chunk % n_cond

    @functools.partial(
        pl.kernel,
        out_type=jax.ShapeDtypeStruct((2 * n_rows_total, 16), jnp.float32),
        mesh=mesh,
        scratch_types=[
            pltpu.VMEM((epw // 128, 128), jnp.int32),
            pltpu.VMEM(((n_cond + chunk) * 32,), jnp.float32),
            pltpu.VMEM((epc, 16), jnp.float32),
            pltpu.SemaphoreType.DMA,
        ],
        compiler_params=pltpu.CompilerParams(use_tc_tiling_on_sc=False),
    )
    def sc_kernel(ids_hbm, table_hbm, pat_hbm, out_hbm, idx_v, pat_v, gbuf,
                  sem):
        wid = lax.axis_index("s") * n_cores + lax.axis_index("c")
        pltpu.sync_copy(ids_hbm.at[wid], idx_v)
        pltpu.sync_copy(pat_hbm, pat_v)

        def chunk_body(g, carry):
            d1 = pltpu.async_copy(table_hbm.at[idx_v.at[2 * g]],
                                  gbuf.at[pl.ds(0, 128)], sem)
            d2 = pltpu.async_copy(table_hbm.at[idx_v.at[2 * g + 1]],
                                  gbuf.at[pl.ds(128, 128)], sem)
            d1.wait()
            d2.wait()

            p = lax.rem(g * phase_step, n_cond) * 32

            def add_body(rr, c2):
                plsc.addupdate(gbuf.at[rr, pl.ds(0, 16)],
                               pat_v[pl.ds(p + rr * 16, 16)])
                return c2

            lax.fori_loop(0, epc, add_body, 0, unroll=8)

            pltpu.sync_copy(gbuf, out_hbm.at[pl.ds(wid * epw + g * epc, epc)])
            return carry

        lax.fori_loop(0, n_chunks_per_worker, chunk_body, 0)

    return sc_kernel


def kernel(cat_ids, cond_table, cat_table):
    b, n_cond = cat_ids.shape
    dim = cat_table.shape[1]
    n_rows_table = cat_table.shape[0]
    n_cat = n_rows_table // n_cond

    info = plsc.get_sparse_core_info()
    n_cores, n_subcores = info.num_cores, info.num_subcores
    n_workers = n_cores * n_subcores

    n_rows = b * n_cond
    chunk = 128
    rows_per_worker = n_rows // n_workers
    n_chunks_per_worker = rows_per_worker // chunk
    assert rows_per_worker % chunk == 0

    wpb = _TBLK // 4
    wide_rows = wpb * ((n_rows_table + _TBLK - 1) // _TBLK)

    wide = _tc_transpose(cat_table.T, n_rows_table, wide_rows)
    tbl16 = wide.reshape(wide_rows * 8, 16)

    offsets = jnp.arange(n_cond, dtype=jnp.int32) * n_cat
    r = (cat_ids.astype(jnp.int32) + offsets[None, :]).reshape(-1)
    e0 = ((r // _TBLK) * wpb + (r % wpb)) * 8 + ((r % _TBLK) // wpb) * 2
    ids_dbl = (e0[:, None] + jnp.arange(2, dtype=jnp.int32)[None, :]).reshape(
        n_workers, 2 * rows_per_worker // 128, 128)

    reps = (n_cond + chunk + n_cond - 1) // n_cond
    pat = jnp.tile(cond_table[1:n_cond + 1],
                   (reps, 1)).reshape(-1)[:(n_cond + chunk) * dim]

    sc_gather = _make_sc_gather(n_rows, n_cond, chunk, n_chunks_per_worker,
                                n_workers, n_cores)
    out = sc_gather(ids_dbl, tbl16, pat)
    return out.reshape(b, n_cond, dim)


# speedup vs baseline: 2.0672x; 1.0941x over previous
import functools

import jax
import jax.numpy as jnp
from jax import lax
from jax.experimental import pallas as pl
from jax.experimental.pallas import tpu as pltpu
from jax.experimental.pallas import tpu_sc as plsc


_TBLK = 8192


def _tc_transpose(table_t, n_rows, wide_rows):
    n_blocks = (n_rows + _TBLK - 1) // _TBLK
    wpb = _TBLK // 4

    def body(in_ref, out_ref):
        x = in_ref[...]
        eye = jnp.float32(
            lax.broadcasted_iota(jnp.int32, (32, 32), 0) ==
            lax.broadcasted_iota(jnp.int32, (32, 32), 1))
        y = lax.dot_general(x, eye, (((0,), (0,)), ((), ())),
                            preferred_element_type=jnp.float32)
        for k in range(4):
            out_ref[:, k * 32:(k + 1) * 32] = y[k * wpb:(k + 1) * wpb, :]

    return pl.pallas_call(
        body,
        grid=(n_blocks,),
        in_specs=[pl.BlockSpec((32, _TBLK), lambda i: (0, i))],
        out_specs=pl.BlockSpec((wpb, 128), lambda i: (i, 0)),
        out_shape=jax.ShapeDtypeStruct((wide_rows, 128), jnp.float32),
    )(table_t)


def _make_sc_gather(n_rows_total, n_cond, chunk, n_chunks_per_worker,
                    n_workers, n_cores):
    mesh = plsc.VectorSubcoreMesh(core_axis_name="c", subcore_axis_name="s")
    rows_per_worker = chunk * n_chunks_per_worker
    epw = 2 * rows_per_worker
    epc = 2 * chunk
    phase_step = chunk % n_cond

    @functools.partial(
        pl.kernel,
        out_type=jax.ShapeDtypeStruct((2 * n_rows_total, 16), jnp.float32),
        mesh=mesh,
        scratch_types=[
            pltpu.VMEM((epw // 128, 128), jnp.int32),
            pltpu.VMEM(((n_cond + chunk) * 32,), jnp.float32),
            pltpu.VMEM((epc, 16), jnp.float32),
            pltpu.VMEM((epc, 16), jnp.float32),
            pltpu.SemaphoreType.DMA,
            pltpu.SemaphoreType.DMA,
        ],
        compiler_params=pltpu.CompilerParams(use_tc_tiling_on_sc=False),
    )
    def sc_kernel(ids_hbm, table_hbm, pat_hbm, out_hbm, idx_v, pat_v, gbuf0,
                  gbuf1, sem0, sem1):
        wid = lax.axis_index("s") * n_cores + lax.axis_index("c")
        pltpu.sync_copy(ids_hbm.at[wid], idx_v)
        pltpu.sync_copy(pat_hbm, pat_v)

        def fire(g, gb, sem):
            pltpu.async_copy(table_hbm.at[idx_v.at[2 * g]],
                             gb.at[pl.ds(0, 128)], sem)
            pltpu.async_copy(table_hbm.at[idx_v.at[2 * g + 1]],
                             gb.at[pl.ds(128, 128)], sem)

        def drain(g, gb, sem):
            pltpu.make_async_copy(table_hbm.at[idx_v.at[2 * g]],
                                  gb.at[pl.ds(0, 128)], sem).wait()
            pltpu.make_async_copy(table_hbm.at[idx_v.at[2 * g + 1]],
                                  gb.at[pl.ds(128, 128)], sem).wait()

        def add_and_writeback(g, gb):
            p = lax.rem(g * phase_step, n_cond) * 32

            def add_body(rr, c2):
                plsc.addupdate(gb.at[rr, pl.ds(0, 16)],
                               pat_v[pl.ds(p + rr * 16, 16)])
                return c2

            lax.fori_loop(0, epc, add_body, 0, unroll=8)
            pltpu.sync_copy(gb, out_hbm.at[pl.ds(wid * epw + g * epc, epc)])

        fire(0, gbuf0, sem0)

        def pair_body(pr, carry):
            g0 = 2 * pr
            g1 = g0 + 1
            fire(g1, gbuf1, sem1)
            drain(g0, gbuf0, sem0)
            add_and_writeback(g0, gbuf0)

            @pl.when(g1 + 1 < n_chunks_per_worker)
            def _():
                fire(g1 + 1, gbuf0, sem0)

            drain(g1, gbuf1, sem1)
            add_and_writeback(g1, gbuf1)
            return carry

        lax.fori_loop(0, n_chunks_per_worker // 2, pair_body, 0)

    return sc_kernel


def kernel(cat_ids, cond_table, cat_table):
    b, n_cond = cat_ids.shape
    dim = cat_table.shape[1]
    n_rows_table = cat_table.shape[0]
    n_cat = n_rows_table // n_cond

    info = plsc.get_sparse_core_info()
    n_cores, n_subcores = info.num_cores, info.num_subcores
    n_workers = n_cores * n_subcores

    n_rows = b * n_cond
    chunk = 128
    rows_per_worker = n_rows // n_workers
    n_chunks_per_worker = rows_per_worker // chunk
    assert rows_per_worker % chunk == 0

    wpb = _TBLK // 4
    wide_rows = wpb * ((n_rows_table + _TBLK - 1) // _TBLK)

    wide = _tc_transpose(cat_table.T, n_rows_table, wide_rows)
    tbl16 = wide.reshape(wide_rows * 8, 16)

    offsets = jnp.arange(n_cond, dtype=jnp.int32) * n_cat
    r = (cat_ids.astype(jnp.int32) + offsets[None, :]).reshape(-1)
    e0 = ((r // _TBLK) * wpb + (r % wpb)) * 8 + ((r % _TBLK) // wpb) * 2
    ids_dbl = (e0[:, None] + jnp.arange(2, dtype=jnp.int32)[None, :]).reshape(
        n_workers, 2 * rows_per_worker // 128, 128)

    reps = (n_cond + chunk + n_cond - 1) // n_cond
    pat = jnp.tile(cond_table[1:n_cond + 1],
                   (reps, 1)).reshape(-1)[:(n_cond + chunk) * dim]

    sc_gather = _make_sc_gather(n_rows, n_cond, chunk, n_chunks_per_worker,
                                n_workers, n_cores)
    out = sc_gather(ids_dbl, tbl16, pat)
    return out.reshape(b, n_cond, dim)


# speedup vs baseline: 2.8266x; 1.3674x over previous
import functools

import jax
import jax.numpy as jnp
from jax import lax
from jax.experimental import pallas as pl
from jax.experimental.pallas import tpu as pltpu
from jax.experimental.pallas import tpu_sc as plsc


_TBLK = 8192


def _tc_transpose(table_t, n_rows, wide_rows):
    n_blocks = (n_rows + _TBLK - 1) // _TBLK
    wpb = _TBLK // 4

    def body(in_ref, out_ref):
        x = in_ref[...]
        x4 = jnp.concatenate([x[:, k * wpb:(k + 1) * wpb] for k in range(4)],
                             axis=0)
        eye = jnp.float32(
            lax.broadcasted_iota(jnp.int32, (128, 128), 0) ==
            lax.broadcasted_iota(jnp.int32, (128, 128), 1))
        out_ref[...] = lax.dot_general(x4, eye, (((0,), (0,)), ((), ())),
                                       preferred_element_type=jnp.float32)

    return pl.pallas_call(
        body,
        grid=(n_blocks,),
        in_specs=[pl.BlockSpec((32, _TBLK), lambda i: (0, i))],
        out_specs=pl.BlockSpec((wpb, 128), lambda i: (i, 0)),
        out_shape=jax.ShapeDtypeStruct((wide_rows, 128), jnp.float32),
    )(table_t)


def _make_sc_gather(n_rows_total, n_cond, chunk, n_chunks_per_worker,
                    n_workers, n_cores):
    mesh = plsc.VectorSubcoreMesh(core_axis_name="c", subcore_axis_name="s")
    rows_per_worker = chunk * n_chunks_per_worker
    epw = 2 * rows_per_worker
    epc = 2 * chunk
    phase_step = chunk % n_cond

    @functools.partial(
        pl.kernel,
        out_type=jax.ShapeDtypeStruct((2 * n_rows_total, 16), jnp.float32),
        mesh=mesh,
        scratch_types=[
            pltpu.VMEM((epw // 128, 128), jnp.int32),
            pltpu.VMEM(((n_cond + chunk) * 32,), jnp.float32),
            pltpu.VMEM((epc, 16), jnp.float32),
            pltpu.VMEM((epc, 16), jnp.float32),
            pltpu.SemaphoreType.DMA,
            pltpu.SemaphoreType.DMA,
        ],
        compiler_params=pltpu.CompilerParams(use_tc_tiling_on_sc=False),
    )
    def sc_kernel(ids_hbm, table_hbm, pat_hbm, out_hbm, idx_v, pat_v, gbuf0,
                  gbuf1, sem0, sem1):
        wid = lax.axis_index("s") * n_cores + lax.axis_index("c")
        pltpu.sync_copy(ids_hbm.at[wid], idx_v)
        pltpu.sync_copy(pat_hbm, pat_v)

        def fire(g, gb, sem):
            pltpu.async_copy(table_hbm.at[idx_v.at[2 * g]],
                             gb.at[pl.ds(0, 128)], sem)
            pltpu.async_copy(table_hbm.at[idx_v.at[2 * g + 1]],
                             gb.at[pl.ds(128, 128)], sem)

        def drain(g, gb, sem):
            pltpu.make_async_copy(table_hbm.at[idx_v.at[2 * g]],
                                  gb.at[pl.ds(0, 128)], sem).wait()
            pltpu.make_async_copy(table_hbm.at[idx_v.at[2 * g + 1]],
                                  gb.at[pl.ds(128, 128)], sem).wait()

        def add_and_writeback(g, gb):
            p = lax.rem(g * phase_step, n_cond) * 32

            def add_body(rr, c2):
                plsc.addupdate(gb.at[rr, pl.ds(0, 16)],
                               pat_v[pl.ds(p + rr * 16, 16)])
                return c2

            lax.fori_loop(0, epc, add_body, 0, unroll=8)
            pltpu.sync_copy(gb, out_hbm.at[pl.ds(wid * epw + g * epc, epc)])

        fire(0, gbuf0, sem0)

        def pair_body(pr, carry):
            g0 = 2 * pr
            g1 = g0 + 1
            fire(g1, gbuf1, sem1)
            drain(g0, gbuf0, sem0)
            add_and_writeback(g0, gbuf0)

            @pl.when(g1 + 1 < n_chunks_per_worker)
            def _():
                fire(g1 + 1, gbuf0, sem0)

            drain(g1, gbuf1, sem1)
            add_and_writeback(g1, gbuf1)
            return carry

        lax.fori_loop(0, n_chunks_per_worker // 2, pair_body, 0)

    return sc_kernel


def kernel(cat_ids, cond_table, cat_table):
    b, n_cond = cat_ids.shape
    dim = cat_table.shape[1]
    n_rows_table = cat_table.shape[0]
    n_cat = n_rows_table // n_cond

    info = plsc.get_sparse_core_info()
    n_cores, n_subcores = info.num_cores, info.num_subcores
    n_workers = n_cores * n_subcores

    n_rows = b * n_cond
    chunk = 128
    rows_per_worker = n_rows // n_workers
    n_chunks_per_worker = rows_per_worker // chunk
    assert rows_per_worker % chunk == 0

    wpb = _TBLK // 4
    wide_rows = wpb * ((n_rows_table + _TBLK - 1) // _TBLK)

    wide = _tc_transpose(cat_table.T, n_rows_table, wide_rows)
    tbl16 = wide.reshape(wide_rows * 8, 16)

    offsets = jnp.arange(n_cond, dtype=jnp.int32) * n_cat
    r = (cat_ids.astype(jnp.int32) + offsets[None, :]).reshape(-1)
    e0 = ((r // _TBLK) * wpb + (r % wpb)) * 8 + ((r % _TBLK) // wpb) * 2
    ids_dbl = (e0[:, None] + jnp.arange(2, dtype=jnp.int32)[None, :]).reshape(
        n_workers, 2 * rows_per_worker // 128, 128)

    reps = (n_cond + chunk + n_cond - 1) // n_cond
    pat = jnp.tile(cond_table[1:n_cond + 1],
                   (reps, 1)).reshape(-1)[:(n_cond + chunk) * dim]

    sc_gather = _make_sc_gather(n_rows, n_cond, chunk, n_chunks_per_worker,
                                n_workers, n_cores)
    out = sc_gather(ids_dbl, tbl16, pat)
    return out.reshape(b, n_cond, dim)


# speedup vs baseline: 3.2444x; 1.1478x over previous
import functools

import jax
import jax.numpy as jnp
from jax import lax
from jax.experimental import pallas as pl
from jax.experimental.pallas import tpu as pltpu
from jax.experimental.pallas import tpu_sc as plsc


_TBLK = 16384


def _tc_transpose(table_t, n_rows, wide_rows):
    n_blocks = (n_rows + _TBLK - 1) // _TBLK
    wpb = _TBLK // 4

    def body(in_ref, out_ref):
        x = in_ref[...]
        x4 = jnp.concatenate([x[:, k * wpb:(k + 1) * wpb] for k in range(4)],
                             axis=0)
        eye = jnp.float32(
            lax.broadcasted_iota(jnp.int32, (128, 128), 0) ==
            lax.broadcasted_iota(jnp.int32, (128, 128), 1))
        out_ref[...] = lax.dot_general(x4, eye, (((0,), (0,)), ((), ())),
                                       preferred_element_type=jnp.float32)

    return pl.pallas_call(
        body,
        grid=(n_blocks,),
        in_specs=[pl.BlockSpec((32, _TBLK), lambda i: (0, i))],
        out_specs=pl.BlockSpec((wpb, 128), lambda i: (i, 0)),
        out_shape=jax.ShapeDtypeStruct((wide_rows, 128), jnp.float32),
    )(table_t)


def _make_sc_gather(n_rows_total, n_cond, chunk, n_chunks_per_worker,
                    n_workers, n_cores):
    mesh = plsc.VectorSubcoreMesh(core_axis_name="c", subcore_axis_name="s")
    rows_per_worker = chunk * n_chunks_per_worker
    epw = 2 * rows_per_worker
    epc = 2 * chunk
    phase_step = chunk % n_cond

    @functools.partial(
        pl.kernel,
        out_type=jax.ShapeDtypeStruct((2 * n_rows_total, 16), jnp.float32),
        mesh=mesh,
        scratch_types=[
            pltpu.VMEM((epw // 128, 128), jnp.int32),
            pltpu.VMEM(((n_cond + chunk) * 32,), jnp.float32),
            pltpu.VMEM((epc, 16), jnp.float32),
            pltpu.VMEM((epc, 16), jnp.float32),
            pltpu.SemaphoreType.DMA,
            pltpu.SemaphoreType.DMA,
        ],
        compiler_params=pltpu.CompilerParams(use_tc_tiling_on_sc=False),
    )
    def sc_kernel(ids_hbm, table_hbm, pat_hbm, out_hbm, idx_v, pat_v, gbuf0,
                  gbuf1, sem0, sem1):
        wid = lax.axis_index("s") * n_cores + lax.axis_index("c")
        pltpu.sync_copy(ids_hbm.at[wid], idx_v)
        pltpu.sync_copy(pat_hbm, pat_v)

        def fire(g, gb, sem):
            pltpu.async_copy(table_hbm.at[idx_v.at[2 * g]],
                             gb.at[pl.ds(0, 128)], sem)
            pltpu.async_copy(table_hbm.at[idx_v.at[2 * g + 1]],
                             gb.at[pl.ds(128, 128)], sem)

        def drain(g, gb, sem):
            pltpu.make_async_copy(table_hbm.at[idx_v.at[2 * g]],
                                  gb.at[pl.ds(0, 128)], sem).wait()
            pltpu.make_async_copy(table_hbm.at[idx_v.at[2 * g + 1]],
                                  gb.at[pl.ds(128, 128)], sem).wait()

        def add_and_writeback(g, gb):
            p = lax.rem(g * phase_step, n_cond) * 32

            def add_body(rr, c2):
                plsc.addupdate(gb.at[rr, pl.ds(0, 16)],
                               pat_v[pl.ds(p + rr * 16, 16)])
                return c2

            lax.fori_loop(0, epc, add_body, 0, unroll=8)
            pltpu.sync_copy(gb, out_hbm.at[pl.ds(wid * epw + g * epc, epc)])

        fire(0, gbuf0, sem0)

        def pair_body(pr, carry):
            g0 = 2 * pr
            g1 = g0 + 1
            fire(g1, gbuf1, sem1)
            drain(g0, gbuf0, sem0)
            add_and_writeback(g0, gbuf0)

            @pl.when(g1 + 1 < n_chunks_per_worker)
            def _():
                fire(g1 + 1, gbuf0, sem0)

            drain(g1, gbuf1, sem1)
            add_and_writeback(g1, gbuf1)
            return carry

        lax.fori_loop(0, n_chunks_per_worker // 2, pair_body, 0)

    return sc_kernel


def kernel(cat_ids, cond_table, cat_table):
    b, n_cond = cat_ids.shape
    dim = cat_table.shape[1]
    n_rows_table = cat_table.shape[0]
    n_cat = n_rows_table // n_cond

    info = plsc.get_sparse_core_info()
    n_cores, n_subcores = info.num_cores, info.num_subcores
    n_workers = n_cores * n_subcores

    n_rows = b * n_cond
    chunk = 128
    rows_per_worker = n_rows // n_workers
    n_chunks_per_worker = rows_per_worker // chunk
    assert rows_per_worker % chunk == 0

    wpb = _TBLK // 4
    wide_rows = wpb * ((n_rows_table + _TBLK - 1) // _TBLK)

    wide = _tc_transpose(cat_table.T, n_rows_table, wide_rows)
    tbl16 = wide.reshape(wide_rows * 8, 16)

    offsets = jnp.arange(n_cond, dtype=jnp.int32) * n_cat
    r = (cat_ids.astype(jnp.int32) + offsets[None, :]).reshape(-1)
    e0 = ((r // _TBLK) * wpb + (r % wpb)) * 8 + ((r % _TBLK) // wpb) * 2
    ids_dbl = (e0[:, None] + jnp.arange(2, dtype=jnp.int32)[None, :]).reshape(
        n_workers, 2 * rows_per_worker // 128, 128)

    reps = (n_cond + chunk + n_cond - 1) // n_cond
    pat = jnp.tile(cond_table[1:n_cond + 1],
                   (reps, 1)).reshape(-1)[:(n_cond + chunk) * dim]

    sc_gather = _make_sc_gather(n_rows, n_cond, chunk, n_chunks_per_worker,
                                n_workers, n_cores)
    out = sc_gather(ids_dbl, tbl16, pat)
    return out.reshape(b, n_cond, dim)


# speedup vs baseline: 3.4278x; 1.0565x over previous
import functools

import jax
import jax.numpy as jnp
from jax import lax
from jax.experimental import pallas as pl
from jax.experimental.pallas import tpu as pltpu
from jax.experimental.pallas import tpu_sc as plsc


_TBLK = 32768


def _tc_transpose(table_t, n_rows, wide_rows):
    n_blocks = (n_rows + _TBLK - 1) // _TBLK
    wpb = _TBLK // 4

    def body(in_ref, out_ref):
        x = in_ref[...]
        x4 = jnp.concatenate([x[:, k * wpb:(k + 1) * wpb] for k in range(4)],
                             axis=0)
        eye = jnp.float32(
            lax.broadcasted_iota(jnp.int32, (128, 128), 0) ==
            lax.broadcasted_iota(jnp.int32, (128, 128), 1))
        out_ref[...] = lax.dot_general(x4, eye, (((0,), (0,)), ((), ())),
                                       preferred_element_type=jnp.float32)

    return pl.pallas_call(
        body,
        grid=(n_blocks,),
        in_specs=[pl.BlockSpec((32, _TBLK), lambda i: (0, i))],
        out_specs=pl.BlockSpec((wpb, 128), lambda i: (i, 0)),
        out_shape=jax.ShapeDtypeStruct((wide_rows, 128), jnp.float32),
    )(table_t)


def _make_sc_gather(n_rows_total, n_cond, chunk, n_chunks_per_worker,
                    n_workers, n_cores):
    mesh = plsc.VectorSubcoreMesh(core_axis_name="c", subcore_axis_name="s")
    rows_per_worker = chunk * n_chunks_per_worker
    epw = 2 * rows_per_worker
    epc = 2 * chunk
    phase_step = chunk % n_cond

    @functools.partial(
        pl.kernel,
        out_type=jax.ShapeDtypeStruct((2 * n_rows_total, 16), jnp.float32),
        mesh=mesh,
        scratch_types=[
            pltpu.VMEM((epw // 128, 128), jnp.int32),
            pltpu.VMEM(((n_cond + chunk) * 32,), jnp.float32),
            pltpu.VMEM((epc, 16), jnp.float32),
            pltpu.VMEM((epc, 16), jnp.float32),
            pltpu.SemaphoreType.DMA,
            pltpu.SemaphoreType.DMA,
        ],
        compiler_params=pltpu.CompilerParams(use_tc_tiling_on_sc=False),
    )
    def sc_kernel(ids_hbm, table_hbm, pat_hbm, out_hbm, idx_v, pat_v, gbuf0,
                  gbuf1, sem0, sem1):
        wid = lax.axis_index("s") * n_cores + lax.axis_index("c")
        pltpu.sync_copy(ids_hbm.at[wid], idx_v)
        pltpu.sync_copy(pat_hbm, pat_v)

        def fire(g, gb, sem):
            pltpu.async_copy(table_hbm.at[idx_v.at[2 * g]],
                             gb.at[pl.ds(0, 128)], sem)
            pltpu.async_copy(table_hbm.at[idx_v.at[2 * g + 1]],
                             gb.at[pl.ds(128, 128)], sem)

        def drain(g, gb, sem):
            pltpu.make_async_copy(table_hbm.at[idx_v.at[2 * g]],
                                  gb.at[pl.ds(0, 128)], sem).wait()
            pltpu.make_async_copy(table_hbm.at[idx_v.at[2 * g + 1]],
                                  gb.at[pl.ds(128, 128)], sem).wait()

        def add_and_writeback(g, gb):
            p = lax.rem(g * phase_step, n_cond) * 32

            def add_body(rr, c2):
                plsc.addupdate(gb.at[rr, pl.ds(0, 16)],
                               pat_v[pl.ds(p + rr * 16, 16)])
                return c2

            lax.fori_loop(0, epc, add_body, 0, unroll=8)
            pltpu.sync_copy(gb, out_hbm.at[pl.ds(wid * epw + g * epc, epc)])

        fire(0, gbuf0, sem0)

        def pair_body(pr, carry):
            g0 = 2 * pr
            g1 = g0 + 1
            fire(g1, gbuf1, sem1)
            drain(g0, gbuf0, sem0)
            add_and_writeback(g0, gbuf0)

            @pl.when(g1 + 1 < n_chunks_per_worker)
            def _():
                fire(g1 + 1, gbuf0, sem0)

            drain(g1, gbuf1, sem1)
            add_and_writeback(g1, gbuf1)
            return carry

        lax.fori_loop(0, n_chunks_per_worker // 2, pair_body, 0)

    return sc_kernel


def kernel(cat_ids, cond_table, cat_table):
    b, n_cond = cat_ids.shape
    dim = cat_table.shape[1]
    n_rows_table = cat_table.shape[0]
    n_cat = n_rows_table // n_cond

    info = plsc.get_sparse_core_info()
    n_cores, n_subcores = info.num_cores, info.num_subcores
    n_workers = n_cores * n_subcores

    n_rows = b * n_cond
    chunk = 128
    rows_per_worker = n_rows // n_workers
    n_chunks_per_worker = rows_per_worker // chunk
    assert rows_per_worker % chunk == 0

    wpb = _TBLK // 4
    wide_rows = wpb * ((n_rows_table + _TBLK - 1) // _TBLK)

    wide = _tc_transpose(cat_table.T, n_rows_table, wide_rows)
    tbl16 = wide.reshape(wide_rows * 8, 16)

    offsets = jnp.arange(n_cond, dtype=jnp.int32) * n_cat
    r = (cat_ids.astype(jnp.int32) + offsets[None, :]).reshape(-1)
    e0 = ((r // _TBLK) * wpb + (r % wpb)) * 8 + ((r % _TBLK) // wpb) * 2
    ids_dbl = (e0[:, None] + jnp.arange(2, dtype=jnp.int32)[None, :]).reshape(
        n_workers, 2 * rows_per_worker // 128, 128)

    reps = (n_cond + chunk + n_cond - 1) // n_cond
    pat = jnp.tile(cond_table[1:n_cond + 1],
                   (reps, 1)).reshape(-1)[:(n_cond + chunk) * dim]

    sc_gather = _make_sc_gather(n_rows, n_cond, chunk, n_chunks_per_worker,
                                n_workers, n_cores)
    out = sc_gather(ids_dbl, tbl16, pat)
    return out.reshape(b, n_cond, dim)
